# baseline (speedup 1.0000x reference)
import functools

import jax
import jax.numpy as jnp
import numpy as np
from jax import lax
from jax.experimental import pallas as pl
from jax.experimental.pallas import tpu as pltpu

B = 32
V = 1000000
K = 64
BC = 16384
LBLK = 128
NT = pl.cdiv(V, BC)
VPAD = NT * BC
NBLK = VPAD // LBLK
MBLK = 128
NCAND = MBLK * LBLK
NEG_INF = np.float32(-np.inf)
IBIG = np.int32(2**30)


def _rotl(x, r):
    return (x << np.uint32(r)) | (x >> np.uint32(32 - r))


def _threefry2x32(k0, k1, x0, x1):
    ks2 = k0 ^ k1 ^ np.uint32(0x1BD11BDA)
    ks = [k0, k1, ks2]
    x0 = x0 + k0
    x1 = x1 + k1
    rotations = [[13, 15, 26, 6], [17, 29, 16, 24]]
    for i in range(5):
        for r in rotations[i % 2]:
            x0 = x0 + x1
            x1 = _rotl(x1, r)
            x1 = x1 ^ x0
        x0 = x0 + ks[(i + 1) % 3]
        x1 = x1 + ks[(i + 2) % 3] + np.uint32(i + 1)
    return x0, x1


_K0, _K1 = (lambda o: (np.uint32(o[0]), np.uint32(o[1])))(
    _threefry2x32(np.uint32(0), np.uint32(0), np.uint32(0), np.uint32(1)))


def _scores_from_cols(cols, probs):
    rows = jax.lax.broadcasted_iota(jnp.uint32, cols.shape, 0)
    j = rows * jnp.uint32(V) + cols.astype(jnp.uint32)
    o0, o1 = _threefry2x32(jnp.uint32(_K0), jnp.uint32(_K1),
                           jnp.zeros_like(j), j)
    bits = o0 ^ o1
    f = jax.lax.bitcast_convert_type(
        (bits >> np.uint32(9)) | np.uint32(0x3F800000), jnp.float32) - 1.0
    minval = jnp.float32(1e-12)
    u = jnp.maximum(minval, f * (jnp.float32(1.0) - minval) + minval)
    return jnp.log(u) / probs


def _blockmax_kernel(probs_ref, out_ref):
    pid = pl.program_id(0)
    p = probs_ref[...]
    cols = jax.lax.broadcasted_iota(jnp.int32, p.shape, 1) + pid * BC
    s = _scores_from_cols(cols, p)
    s = jnp.where(cols < V, s, NEG_INF)
    out_ref[...] = jnp.max(s.reshape(B, BC // LBLK, LBLK), axis=2)


def _topblocks_kernel(bmax_ref, c0_ref, vals_ref):
    vals_ref[...] = bmax_ref[...]
    c0_ref[...] = jnp.zeros((B, MBLK), jnp.int32)
    coliota = jax.lax.broadcasted_iota(jnp.int32, (B, NBLK), 1)
    slotiota = jax.lax.broadcasted_iota(jnp.int32, (B, MBLK), 1)

    def body(t, _):
        v = vals_ref[...]
        m = jnp.max(v, axis=1, keepdims=True)
        sel = jnp.min(jnp.where(v == m, coliota, IBIG), axis=1, keepdims=True)
        c0_ref[...] = jnp.where(slotiota == t, sel, c0_ref[...])
        vals_ref[...] = jnp.where(coliota == sel, NEG_INF, v)
        return 0

    lax.fori_loop(0, MBLK, body, 0)

    c0_ref[...] = jnp.minimum(c0_ref[...] * LBLK, jnp.int32(V - LBLK))


def _final_topk_kernel(g_ref, c0_ref, out_ref, s_ref, col_ref):
    c0 = c0_ref[...]
    cols3 = (c0[:, :, None]
             + jax.lax.broadcasted_iota(jnp.int32, (B, MBLK, LBLK), 2))
    cols = cols3.reshape(B, NCAND)
    p = g_ref[...]
    s_ref[...] = _scores_from_cols(cols, p)
    col_ref[...] = cols
    out_ref[...] = jnp.zeros((B, K), jnp.int32)
    slotiota = jax.lax.broadcasted_iota(jnp.int32, (B, K), 1)

    def body(t, _):
        v = s_ref[...]
        c = col_ref[...]
        m = jnp.max(v, axis=1, keepdims=True)
        sel = jnp.min(jnp.where(v == m, c, IBIG), axis=1, keepdims=True)
        out_ref[...] = jnp.where(slotiota == t, sel, out_ref[...])
        s_ref[...] = jnp.where(c == sel, NEG_INF, v)
        return 0

    lax.fori_loop(0, K, body, 0)


def kernel(probs):
    bmax = pl.pallas_call(
        _blockmax_kernel,
        grid=(NT,),
        in_specs=[pl.BlockSpec((B, BC), lambda i: (0, i))],
        out_specs=pl.BlockSpec((B, BC // LBLK), lambda i: (0, i)),
        out_shape=jax.ShapeDtypeStruct((B, NBLK), jnp.float32),
    )(probs)

    c0 = pl.pallas_call(
        _topblocks_kernel,
        out_shape=jax.ShapeDtypeStruct((B, MBLK), jnp.int32),
        scratch_shapes=[pltpu.VMEM((B, NBLK), jnp.float32)],
    )(bmax)

    g = jax.vmap(
        lambda prow, starts: jax.vmap(
            lambda c: lax.dynamic_slice(prow, (c,), (LBLK,)))(starts)
    )(probs, c0).reshape(B, NCAND)

    out = pl.pallas_call(
        _final_topk_kernel,
        out_shape=jax.ShapeDtypeStruct((B, K), jnp.int32),
        scratch_shapes=[
            pltpu.VMEM((B, NCAND), jnp.float32),
            pltpu.VMEM((B, NCAND), jnp.int32),
        ],
    )(g, c0)
    return out


# speedup vs baseline: 11.6339x; 11.6339x over previous
import functools

import jax
import jax.numpy as jnp
import numpy as np
from jax import lax
from jax.experimental import pallas as pl
from jax.experimental.pallas import tpu as pltpu

B = 32
V = 1000000
K = 64
BC = 16384
LBLK = 128
NT = pl.cdiv(V, BC)
VPAD = NT * BC
NBLK = VPAD // LBLK
MBLK = 128
NCAND = MBLK * LBLK
NEG_INF = np.float32(-np.inf)
IBIG = np.int32(2**30)


def _rotl(x, r):
    return (x << np.uint32(r)) | (x >> np.uint32(32 - r))


def _threefry2x32(k0, k1, x0, x1):
    ks2 = k0 ^ k1 ^ np.uint32(0x1BD11BDA)
    ks = [k0, k1, ks2]
    x0 = x0 + k0
    x1 = x1 + k1
    rotations = [[13, 15, 26, 6], [17, 29, 16, 24]]
    for i in range(5):
        for r in rotations[i % 2]:
            x0 = x0 + x1
            x1 = _rotl(x1, r)
            x1 = x1 ^ x0
        x0 = x0 + ks[(i + 1) % 3]
        x1 = x1 + ks[(i + 2) % 3] + np.uint32(i + 1)
    return x0, x1


_K0, _K1 = (lambda o: (np.uint32(o[0]), np.uint32(o[1])))(
    _threefry2x32(np.uint32(0), np.uint32(0), np.uint32(0), np.uint32(1)))


def _scores_from_cols(cols, probs):
    rows = jax.lax.broadcasted_iota(jnp.uint32, cols.shape, 0)
    j = rows * jnp.uint32(V) + cols.astype(jnp.uint32)
    o0, o1 = _threefry2x32(jnp.uint32(_K0), jnp.uint32(_K1),
                           jnp.zeros_like(j), j)
    bits = o0 ^ o1
    f = jax.lax.bitcast_convert_type(
        (bits >> np.uint32(9)) | np.uint32(0x3F800000), jnp.float32) - 1.0
    minval = jnp.float32(1e-12)
    u = jnp.maximum(minval, f * (jnp.float32(1.0) - minval) + minval)
    return jnp.log(u) / probs


def _blockmax_kernel(probs_ref, out_ref):
    pid = pl.program_id(0)
    p = probs_ref[...]
    cols = jax.lax.broadcasted_iota(jnp.int32, p.shape, 1) + pid * BC
    s = _scores_from_cols(cols, p)
    s = jnp.where(cols < V, s, NEG_INF)
    out_ref[...] = jnp.max(s.reshape(B, BC // LBLK, LBLK), axis=2)


def _topblocks_kernel(bmax_ref, c0_ref, vals_ref):
    vals_ref[...] = bmax_ref[...]
    c0_ref[...] = jnp.zeros((B, MBLK), jnp.int32)
    coliota = jax.lax.broadcasted_iota(jnp.int32, (B, NBLK), 1)
    slotiota = jax.lax.broadcasted_iota(jnp.int32, (B, MBLK), 1)

    def body(t, _):
        v = vals_ref[...]
        m = jnp.max(v, axis=1, keepdims=True)
        sel = jnp.min(jnp.where(v == m, coliota, IBIG), axis=1, keepdims=True)
        c0_ref[...] = jnp.where(slotiota == t, sel, c0_ref[...])
        vals_ref[...] = jnp.where(coliota == sel, NEG_INF, v)
        return 0

    lax.fori_loop(0, MBLK, body, 0)

    c0_ref[...] = jnp.minimum(c0_ref[...] * LBLK, jnp.int32(V - LBLK))


def _final_topk_kernel(g_ref, c0_ref, out_ref, s_ref, col_ref):
    c0 = c0_ref[...]
    cols3 = (c0[:, :, None]
             + jax.lax.broadcasted_iota(jnp.int32, (B, MBLK, LBLK), 2))
    cols = cols3.reshape(B, NCAND)
    p = g_ref[...]
    s_ref[...] = _scores_from_cols(cols, p)
    col_ref[...] = cols
    out_ref[...] = jnp.zeros((B, K), jnp.int32)
    slotiota = jax.lax.broadcasted_iota(jnp.int32, (B, K), 1)

    def body(t, _):
        v = s_ref[...]
        c = col_ref[...]
        m = jnp.max(v, axis=1, keepdims=True)
        sel = jnp.min(jnp.where(v == m, c, IBIG), axis=1, keepdims=True)
        out_ref[...] = jnp.where(slotiota == t, sel, out_ref[...])
        s_ref[...] = jnp.where(c == sel, NEG_INF, v)
        return 0

    lax.fori_loop(0, K, body, 0)


def kernel(probs):
    bmax = pl.pallas_call(
        _blockmax_kernel,
        grid=(NT,),
        in_specs=[pl.BlockSpec((B, BC), lambda i: (0, i))],
        out_specs=pl.BlockSpec((B, BC // LBLK), lambda i: (0, i)),
        out_shape=jax.ShapeDtypeStruct((B, NBLK), jnp.float32),
    )(probs)

    c0 = pl.pallas_call(
        _topblocks_kernel,
        out_shape=jax.ShapeDtypeStruct((B, MBLK), jnp.int32),
        scratch_shapes=[pltpu.VMEM((B, NBLK), jnp.float32)],
    )(bmax)

    cols = (c0[:, :, None] + jax.lax.broadcasted_iota(
        jnp.int32, (B, MBLK, LBLK), 2)).reshape(B, NCAND)
    g = jnp.take_along_axis(probs, cols, axis=1)

    out = pl.pallas_call(
        _final_topk_kernel,
        out_shape=jax.ShapeDtypeStruct((B, K), jnp.int32),
        scratch_shapes=[
            pltpu.VMEM((B, NCAND), jnp.float32),
            pltpu.VMEM((B, NCAND), jnp.int32),
        ],
    )(g, c0)
    return out
